# Initial kernel scaffold; baseline (speedup 1.0000x reference)
import jax
import jax.numpy as jnp
from jax.experimental import pallas as pl


def kernel(x, L, W_g1, b_g1, W_g2, b_g2, W_fc1, b_fc1, W_fc2, b_fc2):
    raise NotImplementedError("write your pallas kernel here")


# speedup vs baseline: 1.9107x; 1.9107x over previous
import jax
import jax.numpy as jnp
from jax.experimental import pallas as pl
from jax.experimental.pallas import tpu as pltpu

N = 2048
F = 128
C1 = 32
C2 = 32
KC = 4
BLK = 256

_f32 = jnp.float32
_bf16 = jnp.bfloat16


def _s1_kernel(l0_ref, x_ref, p1_ref, b1_ref, y0_ref):
    l0 = l0_ref[...]
    x0 = x_ref[...]
    x0f = x0.astype(_f32)
    x1f = jnp.dot(l0, x0, preferred_element_type=_f32)
    x1 = x1f.astype(_bf16)
    x2f = 2.0 * jnp.dot(l0, x1, preferred_element_type=_f32) - x0f
    x2 = x2f.astype(_bf16)
    x3f = 2.0 * jnp.dot(l0, x2, preferred_element_type=_f32) - x1f
    x3 = x3f.astype(_bf16)
    m = jnp.concatenate([x0, x1, x2, x3], axis=1)
    p1 = p1_ref[...]
    b1 = b1_ref[...]
    for i in range(4):
        blk = m[i * 512:(i + 1) * 512, :]
        o = jnp.dot(blk, p1, preferred_element_type=_f32) + b1
        y0_ref[i * 512:(i + 1) * 512, :] = jnp.maximum(o, 0.0).astype(_bf16)


def _step1_kernel(l_ref, yfull_ref, o_ref):
    z = jnp.dot(l_ref[...], yfull_ref[...], preferred_element_type=_f32)
    o_ref[...] = z.astype(_bf16)


def _step2_kernel(l_ref, yfull_ref, yprev_ref, o_ref):
    z = jnp.dot(l_ref[...], yfull_ref[...], preferred_element_type=_f32)
    o_ref[...] = (2.0 * z - yprev_ref[...].astype(_f32)).astype(_bf16)


def _final_kernel(l_ref, y2full_ref, y0_ref, y1_ref, y2_ref, p2_ref, b2_ref,
                  wfc_ref, o_ref):
    z = jnp.dot(l_ref[...], y2full_ref[...], preferred_element_type=_f32)
    y3 = (2.0 * z - y1_ref[...].astype(_f32)).astype(_bf16)
    y0 = y0_ref[...]
    y1 = y1_ref[...]
    y2 = y2_ref[...]
    p2 = p2_ref[...]
    b2 = b2_ref[...]
    acc = jnp.zeros((o_ref.shape[0], o_ref.shape[1]), _f32)
    for c in range(F // 4):
        sl = slice(c * 128, (c + 1) * 128)
        cat = jnp.concatenate([y0[:, sl], y1[:, sl], y2[:, sl], y3[:, sl]],
                              axis=1)
        g = jnp.dot(cat, p2, preferred_element_type=_f32) + b2
        g = jnp.maximum(g, 0.0).astype(_bf16)
        acc = acc + jnp.dot(g, wfc_ref[sl, :], preferred_element_type=_f32)
    o_ref[...] = acc


def kernel(x, L, W_g1, b_g1, W_g2, b_g2, W_fc1, b_fc1, W_fc2, b_fc2):
    L0 = L[0].astype(_bf16)
    L2 = L[2].astype(_bf16)
    xb = x.astype(_bf16)

    eyef = jnp.eye(F, dtype=_f32)
    P1 = jnp.einsum('fg,ck->kfgc', eyef, W_g1).reshape(KC * F, F * C1)
    P1 = P1.astype(_bf16)
    b1r = jnp.tile(b_g1, F).reshape(1, F * C1)
    W2km = W_g2.reshape(C2, C1, KC)
    eye4 = jnp.eye(4, dtype=_f32)
    P2 = jnp.einsum('fg,cak->kfagc', eye4, W2km).reshape(4 * 4 * C1, 4 * C2)
    P2 = P2.astype(_bf16)
    b2r = jnp.tile(b_g2, 4).reshape(1, 4 * C2)
    WfcT = (W_fc2 @ W_fc1).T.astype(_bf16)
    bfc = W_fc2 @ b_fc1 + b_fc2

    cp = pltpu.CompilerParams(vmem_limit_bytes=60 * 1024 * 1024)

    y0 = pl.pallas_call(
        _s1_kernel,
        out_shape=jax.ShapeDtypeStruct((N, F * C1), _bf16),
        compiler_params=cp,
    )(L0, xb, P1, b1r)

    nblk = N // BLK
    spec_l = pl.BlockSpec((BLK, N), lambda i: (i, 0))
    spec_full = pl.BlockSpec((N, F * C1), lambda i: (0, 0))
    spec_blk = pl.BlockSpec((BLK, F * C1), lambda i: (i, 0))

    y1 = pl.pallas_call(
        _step1_kernel,
        grid=(nblk,),
        in_specs=[spec_l, spec_full],
        out_specs=spec_blk,
        out_shape=jax.ShapeDtypeStruct((N, F * C1), _bf16),
        compiler_params=cp,
    )(L2, y0)

    y2 = pl.pallas_call(
        _step2_kernel,
        grid=(nblk,),
        in_specs=[spec_l, spec_full, spec_blk],
        out_specs=spec_blk,
        out_shape=jax.ShapeDtypeStruct((N, F * C1), _bf16),
        compiler_params=cp,
    )(L2, y1, y0)

    out = pl.pallas_call(
        _final_kernel,
        grid=(nblk,),
        in_specs=[
            spec_l, spec_full, spec_blk, spec_blk, spec_blk,
            pl.BlockSpec((4 * 4 * C1, 4 * C2), lambda i: (0, 0)),
            pl.BlockSpec((1, 4 * C2), lambda i: (0, 0)),
            pl.BlockSpec((F * C2, 10), lambda i: (0, 0)),
        ],
        out_specs=pl.BlockSpec((BLK, 10), lambda i: (i, 0)),
        out_shape=jax.ShapeDtypeStruct((N, 10), _f32),
        compiler_params=cp,
    )(L2, y2, y0, y1, y2, P2, b2r, WfcT)

    return out + bfc
